# baseline (speedup 1.0000x reference)
import functools

import jax
import jax.numpy as jnp
from jax import lax
from jax.experimental import pallas as pl
from jax.experimental.pallas import tpu as pltpu
from jax.experimental.pallas import tpu_sc as plsc

NC, NS = 2, 16
CH = 128
PAD_ROWS = 128


BATCH = 16


def _sc_scatter(x, src1, dst2):
    N, D = x.shape
    W = NC * NS
    nch = src1.shape[0] // CH // W
    NA = N + PAD_ROWS
    rpw = (N // NS) // 8 * 8
    r_last_extra = N - NS * rpw

    mesh = plsc.VectorSubcoreMesh(core_axis_name="c", subcore_axis_name="s")

    @functools.partial(
        pl.kernel,
        mesh=mesh,
        out_type=jax.ShapeDtypeStruct((NC, NA, D), jnp.float32),
        scratch_types=[
            pltpu.VMEM((2, BATCH * CH), jnp.int32),
            pltpu.VMEM((2 * BATCH, CH), jnp.int32),
            pltpu.VMEM((CH, D), jnp.float32),
            pltpu.VMEM((CH, D), jnp.float32),
            pltpu.VMEM_SHARED((NA, D), jnp.float32),
            pltpu.SemaphoreType.DMA,
            pltpu.SemaphoreType.DMA,
            pltpu.SemaphoreType.DMA,
            pltpu.SemaphoreType.DMA,
        ],
    )
    def k(x_hbm, src_hbm, dst_hbm, out_hbm, sbuf, dbuf, rows0, rows1,
          agg_sh, sem0, sem1, semi0, semi1):
        c = lax.axis_index("c")
        s = lax.axis_index("s")
        w = c * NS + s
        off_e = pl.multiple_of(w * nch * CH, 8)
        off_r = pl.multiple_of(w * nch, 8)

        @pl.loop(0, CH)
        def _(i):
            for j in range(D // 16):
                rows0.at[pl.ds(i, 1), pl.ds(j * 16, 16)][...] = (
                    jnp.zeros((1, 16), jnp.float32))

        row0 = pl.multiple_of(s * rpw, 8)

        def zero_rows(nrows, base_row):
            o = 0
            while o < nrows:
                n = min(CH, nrows - o)
                pltpu.sync_copy(rows0.at[pl.ds(0, n)] if n < CH else rows0,
                                agg_sh.at[pl.ds(pl.multiple_of(base_row + o, 8),
                                                n)])
                o += n

        zero_rows(rpw, row0)

        @pl.when(s == NS - 1)
        def _():
            zero_rows(r_last_extra, row0 + rpw)
        plsc.subcore_barrier()

        nb = nch // BATCH

        def fetch_idx(b, half, sem):
            pltpu.async_copy(
                src_hbm.at[pl.ds(pl.multiple_of(off_e + b * (BATCH * CH), 8),
                                 BATCH * CH)], sbuf.at[half], sem)
            pltpu.async_copy(
                dst_hbm.at[pl.ds(pl.multiple_of(off_r + b * BATCH, 8),
                                 BATCH)],
                dbuf.at[pl.ds(pl.multiple_of(half * BATCH, 8), BATCH)], sem)

        def wait_idx(half, sem):
            pltpu.make_async_copy(
                src_hbm.at[pl.ds(0, BATCH * CH)], sbuf.at[half], sem).wait()
            pltpu.make_async_copy(
                dst_hbm.at[pl.ds(0, BATCH)],
                dbuf.at[pl.ds(pl.multiple_of(half * BATCH, 8), BATCH)],
                sem).wait()

        fetch_idx(0, 0, semi0)

        def process(b, half, sem_mine, sem_other):
            @pl.when(b + 1 < nb)
            def _():
                fetch_idx(b + 1, 1 - half, sem_other)
            wait_idx(half, sem_mine)

            sb = sbuf.at[half]
            bufs = (rows0, sem0), (rows1, sem1)
            g = pltpu.async_copy(x_hbm.at[sb.at[pl.ds(0, CH)]], rows0,
                                 sem0)
            for j in range(BATCH):
                cur, _ = bufs[j % 2]
                g_cur = g
                if j + 1 < BATCH:
                    nxt, nsem = bufs[(j + 1) % 2]
                    g = pltpu.async_copy(
                        x_hbm.at[sb.at[pl.ds((j + 1) * CH, CH)]], nxt,
                        nsem)
                g_cur.wait()
                pltpu.sync_copy(cur, agg_sh.at[dbuf.at[half * BATCH + j]],
                                add=True)

        @pl.loop(0, nb // 2)
        def _(bb):
            process(bb * 2, 0, semi0, semi1)
            process(bb * 2 + 1, 1, semi1, semi0)

        if nb % 2:
            process(nb - 1, 0, semi0, semi1)

        plsc.subcore_barrier()
        pltpu.sync_copy(agg_sh.at[pl.ds(row0, rpw)],
                        out_hbm.at[c].at[pl.ds(row0, rpw)])

        @pl.when(s == NS - 1)
        def _():
            off2 = pl.multiple_of(row0 + rpw, 8)
            pltpu.sync_copy(agg_sh.at[pl.ds(off2, r_last_extra)],
                            out_hbm.at[c].at[pl.ds(off2, r_last_extra)])

    return k(x, src1, dst2)


def _mlp_body(x_ref, agg_ref, w1_ref, b1_ref, g_ref, be_ref, mu_ref,
              var_ref, w2_ref, b2_ref, eps_ref, o_ref):
    eps = eps_ref[0, 0]
    h = (1.0 + eps) * x_ref[...] + agg_ref[0] + agg_ref[1]
    h = lax.dot_general(h, w1_ref[...], (((1,), (1,)), ((), ())),
                        preferred_element_type=jnp.float32)
    h = h + b1_ref[...]
    scale = g_ref[...] * lax.rsqrt(var_ref[...] + 1e-5)
    h = (h - mu_ref[...]) * scale + be_ref[...]
    h = jnp.maximum(h, 0.0)
    h = lax.dot_general(h, w2_ref[...], (((1,), (1,)), ((), ())),
                        preferred_element_type=jnp.float32)
    o_ref[...] = h + b2_ref[...]


def kernel(x, edge_index, W1, b1, gamma, beta, running_mean, running_var,
           W2, b2, eps):
    N, D = x.shape
    E = edge_index.shape[1]
    W = NC * NS

    cpw = -(-E // (W * CH))
    cpw = -(-cpw // BATCH) * BATCH
    e_pad = W * cpw * CH - E
    src = edge_index[0]
    dst = edge_index[1]
    if e_pad:
        src = jnp.concatenate(
            [src, jnp.arange(e_pad, dtype=jnp.int32) % jnp.int32(N)])
        dst = jnp.concatenate(
            [dst, N + (jnp.arange(e_pad, dtype=jnp.int32) % PAD_ROWS)])
    agg2 = _sc_scatter(x, src, dst.reshape(-1, CH))

    R = 1000
    vec = lambda v: v.reshape(1, D)
    full = lambda shp: pl.BlockSpec(shp, lambda i: tuple(0 for _ in shp))
    out = pl.pallas_call(
        _mlp_body,
        grid=(N // R,),
        in_specs=[
            pl.BlockSpec((R, D), lambda i: (i, 0)),
            pl.BlockSpec((NC, R, D), lambda i: (0, i, 0)),
            full((D, D)),
            full((1, D)),
            full((1, D)),
            full((1, D)),
            full((1, D)),
            full((1, D)),
            full((D, D)),
            full((1, D)),
            pl.BlockSpec(memory_space=pltpu.SMEM),
        ],
        out_specs=pl.BlockSpec((R, D), lambda i: (i, 0)),
        out_shape=jax.ShapeDtypeStruct((N, D), jnp.float32),
    )(x, agg2, W1, vec(b1), vec(gamma), vec(beta), vec(running_mean),
      vec(running_var), W2, vec(b2), eps.reshape(1, 1))
    return out


# speedup vs baseline: 1.0205x; 1.0205x over previous
import functools

import jax
import jax.numpy as jnp
from jax import lax
from jax.experimental import pallas as pl
from jax.experimental.pallas import tpu as pltpu
from jax.experimental.pallas import tpu_sc as plsc

NC, NS = 2, 16
CH = 128
PAD_ROWS = 128


BATCH = 16


def _sc_scatter(x, src1, dst2):
    N, D = x.shape
    W = NC * NS
    nch = src1.shape[0] // CH // W
    NA = N + PAD_ROWS
    rpw = (N // NS) // 8 * 8
    r_last_extra = N - NS * rpw

    mesh = plsc.VectorSubcoreMesh(core_axis_name="c", subcore_axis_name="s")

    @functools.partial(
        pl.kernel,
        mesh=mesh,
        out_type=jax.ShapeDtypeStruct((NC, NA, D), jnp.float32),
        scratch_types=[
            pltpu.VMEM((2, BATCH * CH), jnp.int32),
            pltpu.VMEM((2 * BATCH, CH), jnp.int32),
            pltpu.VMEM((CH, D), jnp.float32),
            pltpu.VMEM((CH, D), jnp.float32),
            pltpu.VMEM_SHARED((NA, D), jnp.float32),
            pltpu.SemaphoreType.DMA,
            pltpu.SemaphoreType.DMA,
            pltpu.SemaphoreType.DMA,
            pltpu.SemaphoreType.DMA,
        ],
    )
    def k(x_hbm, src_hbm, dst_hbm, out_hbm, sbuf, dbuf, rows0, rows1,
          agg_sh, sem0, sem1, semi0, semi1):
        c = lax.axis_index("c")
        s = lax.axis_index("s")
        w = c * NS + s
        off_e = pl.multiple_of(w * nch * CH, 8)
        off_r = pl.multiple_of(w * nch, 8)

        @pl.loop(0, CH)
        def _(i):
            for j in range(D // 16):
                rows0.at[pl.ds(i, 1), pl.ds(j * 16, 16)][...] = (
                    jnp.zeros((1, 16), jnp.float32))

        row0 = pl.multiple_of(s * rpw, 8)

        def zero_rows(nrows, base_row):
            o = 0
            while o < nrows:
                n = min(CH, nrows - o)
                pltpu.sync_copy(rows0.at[pl.ds(0, n)] if n < CH else rows0,
                                agg_sh.at[pl.ds(pl.multiple_of(base_row + o, 8),
                                                n)])
                o += n

        zero_rows(rpw, row0)

        @pl.when(s == NS - 1)
        def _():
            zero_rows(r_last_extra, row0 + rpw)
        plsc.subcore_barrier()

        nb = nch // BATCH

        def fetch_idx(b, half, sem):
            pltpu.async_copy(
                src_hbm.at[pl.ds(pl.multiple_of(off_e + b * (BATCH * CH), 8),
                                 BATCH * CH)], sbuf.at[half], sem)
            pltpu.async_copy(
                dst_hbm.at[pl.ds(pl.multiple_of(off_r + b * BATCH, 8),
                                 BATCH)],
                dbuf.at[pl.ds(pl.multiple_of(half * BATCH, 8), BATCH)], sem)

        def wait_idx(half, sem):
            pltpu.make_async_copy(
                src_hbm.at[pl.ds(0, BATCH * CH)], sbuf.at[half], sem).wait()
            pltpu.make_async_copy(
                dst_hbm.at[pl.ds(0, BATCH)],
                dbuf.at[pl.ds(pl.multiple_of(half * BATCH, 8), BATCH)],
                sem).wait()

        fetch_idx(0, 0, semi0)

        def process(b, half, sem_mine, sem_other):
            @pl.when(b + 1 < nb)
            def _():
                fetch_idx(b + 1, 1 - half, sem_other)
            wait_idx(half, sem_mine)

            sb = sbuf.at[half]
            bufs = (rows0, sem0), (rows1, sem1)
            g = pltpu.async_copy(x_hbm.at[sb.at[pl.ds(0, CH)]], rows0,
                                 sem0)
            for j in range(BATCH):
                cur, _ = bufs[j % 2]
                g_cur = g
                if j + 1 < BATCH:
                    nxt, nsem = bufs[(j + 1) % 2]
                    g = pltpu.async_copy(
                        x_hbm.at[sb.at[pl.ds((j + 1) * CH, CH)]], nxt,
                        nsem)
                g_cur.wait()
                pltpu.sync_copy(cur, agg_sh.at[dbuf.at[half * BATCH + j]],
                                add=True)

        @pl.loop(0, nb // 2)
        def _(bb):
            process(bb * 2, 0, semi0, semi1)
            process(bb * 2 + 1, 1, semi1, semi0)

        if nb % 2:
            process(nb - 1, 0, semi0, semi1)

        plsc.subcore_barrier()
        pltpu.sync_copy(agg_sh.at[pl.ds(row0, rpw)],
                        out_hbm.at[c].at[pl.ds(row0, rpw)])

        @pl.when(s == NS - 1)
        def _():
            off2 = pl.multiple_of(row0 + rpw, 8)
            pltpu.sync_copy(agg_sh.at[pl.ds(off2, r_last_extra)],
                            out_hbm.at[c].at[pl.ds(off2, r_last_extra)])

    return k(x, src1, dst2)


def _mlp_body(x_ref, agg_ref, w1_ref, b1_ref, g_ref, be_ref, mu_ref,
              var_ref, w2_ref, b2_ref, eps_ref, o_ref):
    eps = eps_ref[0, 0]
    h = (1.0 + eps) * x_ref[...] + agg_ref[0] + agg_ref[1]
    h = lax.dot_general(h, w1_ref[...], (((1,), (1,)), ((), ())),
                        preferred_element_type=jnp.float32)
    h = h + b1_ref[...]
    scale = g_ref[...] * lax.rsqrt(var_ref[...] + 1e-5)
    h = (h - mu_ref[...]) * scale + be_ref[...]
    h = jnp.maximum(h, 0.0)
    h = lax.dot_general(h, w2_ref[...], (((1,), (1,)), ((), ())),
                        preferred_element_type=jnp.float32)
    o_ref[...] = h + b2_ref[...]


def kernel(x, edge_index, W1, b1, gamma, beta, running_mean, running_var,
           W2, b2, eps):
    N, D = x.shape
    E = edge_index.shape[1]
    W = NC * NS

    cpw = -(-E // (W * CH))
    cpw = -(-cpw // BATCH) * BATCH
    e_pad = W * cpw * CH - E
    src = edge_index[0]
    dst = edge_index[1]
    if e_pad:
        src = jnp.concatenate(
            [src, jnp.arange(e_pad, dtype=jnp.int32) % jnp.int32(N)])
        dst = jnp.concatenate(
            [dst, N + (jnp.arange(e_pad, dtype=jnp.int32) % PAD_ROWS)])
    agg2 = _sc_scatter(x, src, dst.reshape(-1, CH))

    R = 2000
    vec = lambda v: v.reshape(1, D)
    full = lambda shp: pl.BlockSpec(shp, lambda i: tuple(0 for _ in shp))
    out = pl.pallas_call(
        _mlp_body,
        grid=(N // R,),
        in_specs=[
            pl.BlockSpec((R, D), lambda i: (i, 0)),
            pl.BlockSpec((NC, R, D), lambda i: (0, i, 0)),
            full((D, D)),
            full((1, D)),
            full((1, D)),
            full((1, D)),
            full((1, D)),
            full((1, D)),
            full((D, D)),
            full((1, D)),
            pl.BlockSpec(memory_space=pltpu.SMEM),
        ],
        out_specs=pl.BlockSpec((R, D), lambda i: (i, 0)),
        out_shape=jax.ShapeDtypeStruct((N, D), jnp.float32),
    )(x, agg2, W1, vec(b1), vec(gamma), vec(beta), vec(running_mean),
      vec(running_var), W2, vec(b2), eps.reshape(1, 1))
    return out
